# baseline (speedup 1.0000x reference)
import functools

import jax
import jax.numpy as jnp
from jax import lax
from jax.experimental import pallas as pl


A_SZ = 16
N_SEL = 24
K = 2


def _select_body(a_ref, r_ref, p_ref, o_ref):
    a = a_ref[...]
    ap = jnp.max(a, axis=1)
    qi = lax.broadcasted_iota(jnp.int32, (32, 4, 64), 1)
    argq = jnp.min(jnp.where(a == ap[:, None, :], qi, 4), axis=1)
    wi = lax.broadcasted_iota(jnp.int32, (32, 64, 64), 1)
    wj = lax.broadcasted_iota(jnp.int32, (32, 64, 64), 2)
    vi = ap[:, :, None]
    vj = ap[:, None, :]
    beats = (vj > vi) | ((vj == vi) & (wj < wi))
    arank = jnp.sum(beats.astype(jnp.int32), axis=2)
    m4 = ((arank < A_SZ)[:, None, :] & (argq[:, None, :] == qi)).astype(
        jnp.float32
    )

    r = r_ref[...].reshape(8, 32, 4, 64)
    rm = r + m4[None]
    rp = jnp.max(rm, axis=2)
    qi2 = lax.broadcasted_iota(jnp.int32, (8, 32, 4, 64), 2)
    argq2 = jnp.min(jnp.where(rm == rp[:, :, None, :], qi2, 4), axis=2)
    wio = lax.broadcasted_iota(jnp.int32, (8, 32, 64), 2)
    di = argq2 // K
    dj = argq2 % K
    flat = (K * (wio // 8) + di) * 16 + (K * (wio % 8) + dj)

    rpf = rp.reshape(256, 64)
    wi2 = lax.broadcasted_iota(jnp.int32, (256, 64, 64), 1)
    wj2 = lax.broadcasted_iota(jnp.int32, (256, 64, 64), 2)
    vi2 = rpf[:, :, None]
    vj2 = rpf[:, None, :]
    beats2 = (vj2 > vi2) | ((vj2 == vi2) & (wj2 < wi2))
    rrank = jnp.sum(beats2.astype(jnp.int32), axis=2)

    onehot = (rrank[:, None, :] == p_ref[...][None, :, :]).astype(jnp.float32)
    flatf = flat.reshape(256, 64).astype(jnp.float32)
    sel = jnp.sum(flatf[:, None, :] * onehot, axis=2)
    o_ref[...] = sel.astype(jnp.int32)


def _gather_body(v_ref, s_ref, o_ref):
    s = s_ref[...]
    ci = lax.broadcasted_iota(jnp.int32, (256, 32, 256), 2)
    oh = (s[:, :, None] == ci).astype(jnp.float32)
    v = v_ref[...]
    res = lax.dot_general(
        v, oh, (((2,), (2,)), ((0,), (0,))), preferred_element_type=jnp.float32
    )
    o_ref[...] = res[..., :N_SEL]


def _compute_sel(a4, r4, perm2d, interpret=False):
    return pl.pallas_call(
        _select_body,
        grid=(32,),
        in_specs=[
            pl.BlockSpec((32, 4, 64), lambda t: (t // 4, 0, 0)),
            pl.BlockSpec((256, 4, 64), lambda t: (t, 0, 0)),
            pl.BlockSpec((32, 64), lambda t: (0, 0)),
        ],
        out_specs=pl.BlockSpec((256, 32), lambda t: (t, 0)),
        out_shape=jax.ShapeDtypeStruct((8192, 32), jnp.int32),
        interpret=interpret,
    )(a4, r4, perm2d)


def _gather_votes(votes_r, sel, interpret=False):
    return pl.pallas_call(
        _gather_body,
        grid=(32,),
        in_specs=[
            pl.BlockSpec((256, 16, 256), lambda t: (t, 0, 0)),
            pl.BlockSpec((256, 32), lambda t: (t, 0)),
        ],
        out_specs=pl.BlockSpec((256, 16, N_SEL), lambda t: (t, 0, 0)),
        out_shape=jax.ShapeDtypeStruct((8192, 16, N_SEL), jnp.float32),
        interpret=interpret,
    )(votes_r, sel)


@jax.jit
def kernel(x, route, votes):
    b, idim, odim, h, dx, dy = votes.shape
    a_orig = x[..., h - 1]
    a4 = (
        a_orig.reshape(b * odim, dx // K, K, dy // K, K)
        .transpose(0, 2, 4, 1, 3)
        .reshape(b * odim, K * K, (dx // K) * (dy // K))
    )
    r4 = (
        route.reshape(b * idim * odim, dx // K, K, dy // K, K)
        .transpose(0, 2, 4, 1, 3)
        .reshape(b * idim * odim, K * K, (dx // K) * (dy // K))
    )
    perm = jax.random.permutation(jax.random.key(42), N_SEL).astype(jnp.int32)
    perm2d = jnp.broadcast_to(
        jnp.pad(perm, (0, 8), constant_values=-1)[:, None], (32, 64)
    )
    sel = _compute_sel(a4, r4, perm2d)
    votes_r = votes.reshape(b * idim * odim, h, dx * dy)
    out = _gather_votes(votes_r, sel)
    return out.reshape(b, idim, odim, h, N_SEL, 1)
